# baseline (speedup 1.0000x reference)
import functools

import jax
import jax.numpy as jnp
from jax import lax
from jax.experimental import pallas as pl
from jax.experimental.pallas import tpu as pltpu
from jax.experimental.pallas import tpu_sc as plsc

N = 10000
E = 320000
H = 128
T = 3
N_STEPS = 6

NC = 2
NS = 16
NW = NC * NS

K = 128
Q = 80
BLK = 16
NBLK = Q // BLK
EPW = Q * K
E_PAD = EPW * NW

RPT = 632
ACC_ROWS = RPT * NS


def _edge_aggregate(table, gidx4, dst4, zeros_rpt):
    mesh = plsc.VectorSubcoreMesh(core_axis_name="c", subcore_axis_name="s")

    @functools.partial(
        pl.kernel,
        out_type=jax.ShapeDtypeStruct((NC, ACC_ROWS, H), jnp.float32),
        mesh=mesh,
        scratch_types=[
            pltpu.VMEM((BLK, K), jnp.int32),
            pltpu.VMEM((BLK, K), jnp.int32),
            pltpu.VMEM((BLK, K), jnp.int32),
            pltpu.VMEM((BLK, K), jnp.int32),
            pltpu.VMEM((K, H), jnp.float32),
            pltpu.VMEM((K, H), jnp.float32),
            pltpu.VMEM_SHARED((ACC_ROWS, H), jnp.float32),
            pltpu.SemaphoreType.DMA,
            pltpu.SemaphoreType.DMA,
            pltpu.SemaphoreType.DMA,
        ],
    )
    def body(table_hbm, gidx_hbm, dst_hbm, z_hbm, out_hbm,
             g0, g1, d0, d1, rows0, rows1, acc_sh, sem0, sem1, semi):
        cid = lax.axis_index("c")
        sid = lax.axis_index("s")
        wid = cid * NS + sid

        pltpu.sync_copy(gidx_hbm.at[wid, 0], g0)
        pltpu.sync_copy(dst_hbm.at[wid, 0], d0)
        pltpu.sync_copy(z_hbm, acc_sh.at[pl.ds(sid * RPT, RPT)])
        plsc.subcore_barrier()

        gbufs = (g0, g1)
        dbufs = (d0, d1)
        for jb in range(NBLK):
            ga, da = gbufs[jb % 2], dbufs[jb % 2]
            gn, dn = gbufs[1 - jb % 2], dbufs[1 - jb % 2]
            if jb < NBLK - 1:
                pltpu.async_copy(gidx_hbm.at[wid, jb + 1], gn, semi)
                pltpu.async_copy(dst_hbm.at[wid, jb + 1], dn, semi)

            pltpu.async_copy(table_hbm.at[ga.at[0]], rows0, sem0)

            def pair(j2, _, ga=ga, da=da):
                a = 2 * j2
                b = a + 1
                pltpu.async_copy(table_hbm.at[ga.at[b]], rows1, sem1)
                pltpu.make_async_copy(table_hbm.at[ga.at[a]], rows0, sem0).wait()
                pltpu.sync_copy(rows0, acc_sh.at[da.at[a]], add=True)

                @pl.when(j2 < BLK // 2 - 1)
                def _():
                    pltpu.async_copy(table_hbm.at[ga.at[a + 2]], rows0, sem0)

                pltpu.make_async_copy(table_hbm.at[ga.at[b]], rows1, sem1).wait()
                pltpu.sync_copy(rows1, acc_sh.at[da.at[b]], add=True)
                return 0

            lax.fori_loop(0, BLK // 2, pair, 0)
            if jb < NBLK - 1:
                pltpu.make_async_copy(gidx_hbm.at[wid, jb + 1], gn, semi).wait()
                pltpu.make_async_copy(dst_hbm.at[wid, jb + 1], dn, semi).wait()

        plsc.subcore_barrier()
        pltpu.sync_copy(acc_sh.at[pl.ds(sid * RPT, RPT)],
                        out_hbm.at[cid, pl.ds(sid * RPT, RPT)])

    return body(table, gidx4, dst4, zeros_rpt)


def _tc_gather_index(et2, src2):
    BR = 256
    rows = E_PAD // K

    def body(et_ref, src_ref, o_ref):
        o_ref[...] = et_ref[...] * N + src_ref[...]

    return pl.pallas_call(
        body,
        grid=(rows // BR,),
        in_specs=[
            pl.BlockSpec((BR, K), lambda i: (i, 0)),
            pl.BlockSpec((BR, K), lambda i: (i, 0)),
        ],
        out_specs=pl.BlockSpec((BR, K), lambda i: (i, 0)),
        out_shape=jax.ShapeDtypeStruct((rows, K), jnp.int32),
    )(et2, src2)


def _tc_embed(type_ids, emb_table):
    def body(ids_ref, emb_ref, o_ref):
        ids = ids_ref[...]
        onehot = (ids[:, None] == lax.broadcasted_iota(jnp.int32, (N, 128), 1)
                  ).astype(jnp.float32)
        o_ref[...] = jnp.dot(onehot, emb_ref[...],
                             preferred_element_type=jnp.float32)

    emb_pad = jnp.zeros((128, H), jnp.float32).at[:100].set(emb_table)
    return pl.pallas_call(
        body,
        out_shape=jax.ShapeDtypeStruct((N, H), jnp.float32),
    )(type_ids, emb_pad)


def _tc_transform(h, W_e, b_e3):
    BN = 1000

    def body(h_ref, w_ref, b_ref, o_ref):
        o_ref[...] = (jnp.dot(h_ref[...], w_ref[0],
                              preferred_element_type=jnp.float32)
                      + b_ref[0])

    nb = N // BN
    return pl.pallas_call(
        body,
        grid=(T, nb),
        in_specs=[
            pl.BlockSpec((BN, H), lambda t, i: (i, 0)),
            pl.BlockSpec((1, H, H), lambda t, i: (t, 0, 0)),
            pl.BlockSpec((1, 1, H), lambda t, i: (t, 0, 0)),
        ],
        out_specs=pl.BlockSpec((BN, H), lambda t, i: (t * nb + i, 0)),
        out_shape=jax.ShapeDtypeStruct((T * N, H), jnp.float32),
    )(h, W_e, b_e3)


def _tc_gru(parts, h, W_ihT, W_hhT, b_ih, b_hh):
    BN = 1000

    def body(p0_ref, p1_ref, h_ref, wi_ref, wh_ref, bi_ref, bh_ref, o_ref):
        a = p0_ref[0] + p1_ref[0]
        hh = h_ref[...]
        gi = jnp.dot(a, wi_ref[...], preferred_element_type=jnp.float32) + bi_ref[...]
        gh = jnp.dot(hh, wh_ref[...], preferred_element_type=jnp.float32) + bh_ref[...]
        r = jax.nn.sigmoid(gi[:, :H] + gh[:, :H])
        z = jax.nn.sigmoid(gi[:, H:2 * H] + gh[:, H:2 * H])
        n = jnp.tanh(gi[:, 2 * H:] + r * gh[:, 2 * H:])
        o_ref[...] = (1.0 - z) * n + z * hh

    return pl.pallas_call(
        body,
        grid=(N // BN,),
        in_specs=[
            pl.BlockSpec((1, BN, H), lambda i: (0, i, 0)),
            pl.BlockSpec((1, BN, H), lambda i: (1, i, 0)),
            pl.BlockSpec((BN, H), lambda i: (i, 0)),
            pl.BlockSpec((H, 3 * H), lambda i: (0, 0)),
            pl.BlockSpec((H, 3 * H), lambda i: (0, 0)),
            pl.BlockSpec((1, 3 * H), lambda i: (0, 0)),
            pl.BlockSpec((1, 3 * H), lambda i: (0, 0)),
        ],
        out_specs=pl.BlockSpec((BN, H), lambda i: (i, 0)),
        out_shape=jax.ShapeDtypeStruct((N, H), jnp.float32),
    )(parts, parts, h, W_ihT, W_hhT, b_ih, b_hh)


def _tc_pool(h, ann, wg1, wg2, b_gate, wo1, wo2, b_out):
    OUT = b_out.shape[-1]

    def body(h_ref, a_ref, wg1_ref, wg2_ref, bg_ref, wo1_ref, wo2_ref, bo_ref,
             o_ref):
        hh = h_ref[...]
        aa = a_ref[...]
        lg = (jnp.dot(hh, wg1_ref[...], preferred_element_type=jnp.float32)
              + jnp.dot(aa, wg2_ref[...], preferred_element_type=jnp.float32)
              + bg_ref[0, 0])
        m = jnp.max(lg)
        e = jnp.exp(lg - m)
        g = e / jnp.sum(e)
        rh = jnp.sum(g * hh, axis=0, keepdims=True)
        ra = jnp.sum(g * aa, axis=0, keepdims=True)
        o_ref[...] = (jnp.dot(rh, wo1_ref[...], preferred_element_type=jnp.float32)
                      + jnp.dot(ra, wo2_ref[...], preferred_element_type=jnp.float32)
                      + bo_ref[...])

    return pl.pallas_call(
        body,
        out_shape=jax.ShapeDtypeStruct((1, OUT), jnp.float32),
    )(h, ann, wg1, wg2, b_gate, wo1, wo2, b_out)


def kernel(edge_index, etypes, type_ids, emb_table, W_e, b_e, W_ih, W_hh,
           b_ih, b_hh, W_gate, b_gate, W_out, b_out):
    src = edge_index[0]
    dst = edge_index[1]
    pad = E_PAD - E
    et2 = jnp.concatenate([etypes, jnp.full((pad,), T - 1, jnp.int32)]
                          ).reshape(E_PAD // K, K)
    src2 = jnp.concatenate([src, jnp.full((pad,), N - 1, jnp.int32)]
                           ).reshape(E_PAD // K, K)
    gidx = _tc_gather_index(et2, src2).reshape(E_PAD)
    sink = N + (jnp.arange(pad, dtype=jnp.int32) % (ACC_ROWS - N))
    dst_p = jnp.concatenate([dst, sink])
    gidx_s, dst_s = lax.sort((gidx, dst_p), num_keys=1)
    gidx4 = gidx_s.reshape(NW, NBLK, BLK, K)
    dst4 = dst_s.reshape(NW, NBLK, BLK, K)
    zeros_rpt = jnp.zeros((RPT, H), jnp.float32)

    W_ihT = W_ih.T
    W_hhT = W_hh.T
    b_ih2 = b_ih.reshape(1, 3 * H)
    b_hh2 = b_hh.reshape(1, 3 * H)
    b_e3 = b_e.reshape(T, 1, H)
    wg1 = W_gate[:H]
    wg2 = W_gate[H:]
    wo1 = W_out[:H]
    wo2 = W_out[H:]
    bg2 = b_gate.reshape(1, 1)
    bo2 = b_out.reshape(1, -1)

    ann = _tc_embed(type_ids, emb_table)
    h = ann
    for _ in range(N_STEPS):
        table = _tc_transform(h, W_e, b_e3)
        parts = _edge_aggregate(table, gidx4, dst4, zeros_rpt)
        h = _tc_gru(parts, h, W_ihT, W_hhT, b_ih2, b_hh2)
    return _tc_pool(h, ann, wg1, wg2, bg2, wo1, wo2, bo2)


# speedup vs baseline: 2.2452x; 2.2452x over previous
import functools

import jax
import jax.numpy as jnp
from jax import lax
from jax.experimental import pallas as pl
from jax.experimental.pallas import tpu as pltpu
from jax.experimental.pallas import tpu_sc as plsc

N = 10000
E = 320000
H = 128
T = 3
N_STEPS = 6

NC = 2
NS = 16
NW = NC * NS

K = 128
Q = 80
BLK = 16
NBLK = Q // BLK
EPW = Q * K
E_PAD = EPW * NW

RPT = 632
ACC_ROWS = RPT * NS


def _edge_aggregate(table, gidx4, dst4, zeros_rpt):
    mesh = plsc.VectorSubcoreMesh(core_axis_name="c", subcore_axis_name="s")

    @functools.partial(
        pl.kernel,
        out_type=jax.ShapeDtypeStruct((NC, ACC_ROWS, H), jnp.float32),
        mesh=mesh,
        scratch_types=[
            pltpu.VMEM((BLK, K), jnp.int32),
            pltpu.VMEM((BLK, K), jnp.int32),
            pltpu.VMEM((BLK, K), jnp.int32),
            pltpu.VMEM((BLK, K), jnp.int32),
            pltpu.VMEM((K, H), jnp.float32),
            pltpu.VMEM((K, H), jnp.float32),
            pltpu.VMEM_SHARED((ACC_ROWS, H), jnp.float32),
            pltpu.SemaphoreType.DMA,
            pltpu.SemaphoreType.DMA,
            pltpu.SemaphoreType.DMA,
        ],
    )
    def body(table_hbm, gidx_hbm, dst_hbm, z_hbm, out_hbm,
             g0, g1, d0, d1, rows0, rows1, acc_sh, sem0, sem1, semi):
        cid = lax.axis_index("c")
        sid = lax.axis_index("s")
        wid = cid * NS + sid

        pltpu.sync_copy(gidx_hbm.at[wid, 0], g0)
        pltpu.sync_copy(dst_hbm.at[wid, 0], d0)
        pltpu.sync_copy(z_hbm, acc_sh.at[pl.ds(sid * RPT, RPT)])
        plsc.subcore_barrier()

        gbufs = (g0, g1)
        dbufs = (d0, d1)
        for jb in range(NBLK):
            ga, da = gbufs[jb % 2], dbufs[jb % 2]
            gn, dn = gbufs[1 - jb % 2], dbufs[1 - jb % 2]
            if jb < NBLK - 1:
                pltpu.async_copy(gidx_hbm.at[wid, jb + 1], gn, semi)
                pltpu.async_copy(dst_hbm.at[wid, jb + 1], dn, semi)

            pltpu.async_copy(table_hbm.at[ga.at[0]], rows0, sem0)

            def pair(j2, _, ga=ga, da=da):
                a = 2 * j2
                b = a + 1
                pltpu.async_copy(table_hbm.at[ga.at[b]], rows1, sem1)
                pltpu.make_async_copy(table_hbm.at[ga.at[a]], rows0, sem0).wait()
                pltpu.sync_copy(rows0, acc_sh.at[da.at[a]], add=True)

                @pl.when(j2 < BLK // 2 - 1)
                def _():
                    pltpu.async_copy(table_hbm.at[ga.at[a + 2]], rows0, sem0)

                pltpu.make_async_copy(table_hbm.at[ga.at[b]], rows1, sem1).wait()
                pltpu.sync_copy(rows1, acc_sh.at[da.at[b]], add=True)
                return 0

            lax.fori_loop(0, BLK // 2, pair, 0)
            if jb < NBLK - 1:
                pltpu.make_async_copy(gidx_hbm.at[wid, jb + 1], gn, semi).wait()
                pltpu.make_async_copy(dst_hbm.at[wid, jb + 1], dn, semi).wait()

        plsc.subcore_barrier()
        pltpu.sync_copy(acc_sh.at[pl.ds(sid * RPT, RPT)],
                        out_hbm.at[cid, pl.ds(sid * RPT, RPT)])

    return body(table, gidx4, dst4, zeros_rpt)


def _tc_gather_index(et2, src2):
    BR = 256
    rows = E_PAD // K

    def body(et_ref, src_ref, o_ref):
        o_ref[...] = et_ref[...] * N + src_ref[...]

    return pl.pallas_call(
        body,
        grid=(rows // BR,),
        in_specs=[
            pl.BlockSpec((BR, K), lambda i: (i, 0)),
            pl.BlockSpec((BR, K), lambda i: (i, 0)),
        ],
        out_specs=pl.BlockSpec((BR, K), lambda i: (i, 0)),
        out_shape=jax.ShapeDtypeStruct((rows, K), jnp.int32),
    )(et2, src2)


def _tc_embed(type_ids, emb_table):
    def body(ids_ref, emb_ref, o_ref):
        ids = ids_ref[...]
        onehot = (ids[:, None] == lax.broadcasted_iota(jnp.int32, (N, 128), 1)
                  ).astype(jnp.float32)
        o_ref[...] = jnp.dot(onehot, emb_ref[...],
                             preferred_element_type=jnp.float32)

    emb_pad = jnp.zeros((128, H), jnp.float32).at[:100].set(emb_table)
    return pl.pallas_call(
        body,
        out_shape=jax.ShapeDtypeStruct((N, H), jnp.float32),
    )(type_ids, emb_pad)


def _tc_transform(h, W_e, b_e3):
    BN = 1000

    def body(h_ref, w_ref, b_ref, o_ref):
        o_ref[...] = (jnp.dot(h_ref[...], w_ref[0],
                              preferred_element_type=jnp.float32)
                      + b_ref[0])

    nb = N // BN
    return pl.pallas_call(
        body,
        grid=(T, nb),
        in_specs=[
            pl.BlockSpec((BN, H), lambda t, i: (i, 0)),
            pl.BlockSpec((1, H, H), lambda t, i: (t, 0, 0)),
            pl.BlockSpec((1, 1, H), lambda t, i: (t, 0, 0)),
        ],
        out_specs=pl.BlockSpec((BN, H), lambda t, i: (t * nb + i, 0)),
        out_shape=jax.ShapeDtypeStruct((T * N, H), jnp.float32),
    )(h, W_e, b_e3)


def _tc_gru(parts, h, W_ihT, W_hhT, b_ih, b_hh):
    BN = 1000

    def body(p0_ref, p1_ref, h_ref, wi_ref, wh_ref, bi_ref, bh_ref, o_ref):
        a = p0_ref[0] + p1_ref[0]
        hh = h_ref[...]
        gi = jnp.dot(a, wi_ref[...], preferred_element_type=jnp.float32) + bi_ref[...]
        gh = jnp.dot(hh, wh_ref[...], preferred_element_type=jnp.float32) + bh_ref[...]
        r = jax.nn.sigmoid(gi[:, :H] + gh[:, :H])
        z = jax.nn.sigmoid(gi[:, H:2 * H] + gh[:, H:2 * H])
        n = jnp.tanh(gi[:, 2 * H:] + r * gh[:, 2 * H:])
        o_ref[...] = (1.0 - z) * n + z * hh

    return pl.pallas_call(
        body,
        grid=(N // BN,),
        in_specs=[
            pl.BlockSpec((1, BN, H), lambda i: (0, i, 0)),
            pl.BlockSpec((1, BN, H), lambda i: (1, i, 0)),
            pl.BlockSpec((BN, H), lambda i: (i, 0)),
            pl.BlockSpec((H, 3 * H), lambda i: (0, 0)),
            pl.BlockSpec((H, 3 * H), lambda i: (0, 0)),
            pl.BlockSpec((1, 3 * H), lambda i: (0, 0)),
            pl.BlockSpec((1, 3 * H), lambda i: (0, 0)),
        ],
        out_specs=pl.BlockSpec((BN, H), lambda i: (i, 0)),
        out_shape=jax.ShapeDtypeStruct((N, H), jnp.float32),
    )(parts, parts, h, W_ihT, W_hhT, b_ih, b_hh)


def _tc_pool(h, ann, wg1, wg2, b_gate, wo1, wo2, b_out):
    OUT = b_out.shape[-1]

    def body(h_ref, a_ref, wg1_ref, wg2_ref, bg_ref, wo1_ref, wo2_ref, bo_ref,
             o_ref):
        hh = h_ref[...]
        aa = a_ref[...]
        lg = (jnp.dot(hh, wg1_ref[...], preferred_element_type=jnp.float32)
              + jnp.dot(aa, wg2_ref[...], preferred_element_type=jnp.float32)
              + bg_ref[0, 0])
        m = jnp.max(lg)
        e = jnp.exp(lg - m)
        g = e / jnp.sum(e)
        rh = jnp.sum(g * hh, axis=0, keepdims=True)
        ra = jnp.sum(g * aa, axis=0, keepdims=True)
        o_ref[...] = (jnp.dot(rh, wo1_ref[...], preferred_element_type=jnp.float32)
                      + jnp.dot(ra, wo2_ref[...], preferred_element_type=jnp.float32)
                      + bo_ref[...])

    return pl.pallas_call(
        body,
        out_shape=jax.ShapeDtypeStruct((1, OUT), jnp.float32),
    )(h, ann, wg1, wg2, b_gate, wo1, wo2, b_out)


def kernel(edge_index, etypes, type_ids, emb_table, W_e, b_e, W_ih, W_hh,
           b_ih, b_hh, W_gate, b_gate, W_out, b_out):
    src = edge_index[0]
    dst = edge_index[1]
    pad = E_PAD - E
    pad_r = jnp.arange(pad, dtype=jnp.int32)
    et2 = jnp.concatenate([etypes, pad_r % T]).reshape(E_PAD // K, K)
    src2 = jnp.concatenate([src, (pad_r * 7919) % N]).reshape(E_PAD // K, K)
    gidx = _tc_gather_index(et2, src2).reshape(E_PAD)
    sink = N + (pad_r % (ACC_ROWS - N))
    dst_p = jnp.concatenate([dst, sink])
    gidx_s, dst_s = lax.sort((gidx, dst_p), num_keys=1)
    gidx4 = (gidx_s.reshape(Q, NW, K).transpose(1, 0, 2)
             .reshape(NW, NBLK, BLK, K))
    dst4 = (dst_s.reshape(Q, NW, K).transpose(1, 0, 2)
            .reshape(NW, NBLK, BLK, K))
    zeros_rpt = jnp.zeros((RPT, H), jnp.float32)

    W_ihT = W_ih.T
    W_hhT = W_hh.T
    b_ih2 = b_ih.reshape(1, 3 * H)
    b_hh2 = b_hh.reshape(1, 3 * H)
    b_e3 = b_e.reshape(T, 1, H)
    wg1 = W_gate[:H]
    wg2 = W_gate[H:]
    wo1 = W_out[:H]
    wo2 = W_out[H:]
    bg2 = b_gate.reshape(1, 1)
    bo2 = b_out.reshape(1, -1)

    ann = _tc_embed(type_ids, emb_table)
    h = ann
    for _ in range(N_STEPS):
        table = _tc_transform(h, W_e, b_e3)
        parts = _edge_aggregate(table, gidx4, dst4, zeros_rpt)
        h = _tc_gru(parts, h, W_ihT, W_hhT, b_ih2, b_hh2)
    return _tc_pool(h, ann, wg1, wg2, bg2, wo1, wo2, bo2)


# speedup vs baseline: 2.4833x; 1.1060x over previous
import functools

import jax
import jax.numpy as jnp
from jax import lax
from jax.experimental import pallas as pl
from jax.experimental.pallas import tpu as pltpu
from jax.experimental.pallas import tpu_sc as plsc

N = 10000
E = 320000
H = 128
T = 3
N_STEPS = 6

NC = 2
NS = 16
NW = NC * NS

K = 128
Q = 80
BLK = 16
NBLK = Q // BLK
EPW = Q * K
E_PAD = EPW * NW

RPT = 632
ACC_ROWS = RPT * NS


def _edge_aggregate(table, gidx4, dst4, zeros_rpt):
    mesh = plsc.VectorSubcoreMesh(core_axis_name="c", subcore_axis_name="s")

    @functools.partial(
        pl.kernel,
        out_type=jax.ShapeDtypeStruct((NC, ACC_ROWS, H), jnp.float32),
        mesh=mesh,
        scratch_types=[
            pltpu.VMEM((BLK, K), jnp.int32),
            pltpu.VMEM((BLK, K), jnp.int32),
            pltpu.VMEM((BLK, K), jnp.int32),
            pltpu.VMEM((BLK, K), jnp.int32),
            pltpu.VMEM((K, H), jnp.float32),
            pltpu.VMEM((K, H), jnp.float32),
            pltpu.VMEM_SHARED((ACC_ROWS, H), jnp.float32),
            pltpu.SemaphoreType.DMA,
            pltpu.SemaphoreType.DMA,
            pltpu.SemaphoreType.DMA,
        ],
    )
    def body(table_hbm, gidx_hbm, dst_hbm, z_hbm, out_hbm,
             g0, g1, d0, d1, rows0, rows1, acc_sh, sem0, sem1, semi):
        cid = lax.axis_index("c")
        sid = lax.axis_index("s")
        wid = cid * NS + sid

        pltpu.sync_copy(gidx_hbm.at[wid, 0], g0)
        pltpu.sync_copy(dst_hbm.at[wid, 0], d0)
        pltpu.sync_copy(z_hbm, acc_sh.at[pl.ds(sid * RPT, RPT)])
        plsc.subcore_barrier()

        gbufs = (g0, g1)
        dbufs = (d0, d1)
        for jb in range(NBLK):
            ga, da = gbufs[jb % 2], dbufs[jb % 2]
            gn, dn = gbufs[1 - jb % 2], dbufs[1 - jb % 2]
            if jb < NBLK - 1:
                pltpu.async_copy(gidx_hbm.at[wid, jb + 1], gn, semi)
                pltpu.async_copy(dst_hbm.at[wid, jb + 1], dn, semi)

            pltpu.async_copy(table_hbm.at[ga.at[0]], rows0, sem0)

            def pair(j2, _, ga=ga, da=da):
                a = 2 * j2
                b = a + 1
                pltpu.async_copy(table_hbm.at[ga.at[b]], rows1, sem1)
                pltpu.make_async_copy(table_hbm.at[ga.at[a]], rows0, sem0).wait()
                pltpu.sync_copy(rows0, acc_sh.at[da.at[a]], add=True)

                @pl.when(j2 < BLK // 2 - 1)
                def _():
                    pltpu.async_copy(table_hbm.at[ga.at[a + 2]], rows0, sem0)

                pltpu.make_async_copy(table_hbm.at[ga.at[b]], rows1, sem1).wait()
                pltpu.sync_copy(rows1, acc_sh.at[da.at[b]], add=True)
                return 0

            lax.fori_loop(0, BLK // 2, pair, 0)
            if jb < NBLK - 1:
                pltpu.make_async_copy(gidx_hbm.at[wid, jb + 1], gn, semi).wait()
                pltpu.make_async_copy(dst_hbm.at[wid, jb + 1], dn, semi).wait()

        plsc.subcore_barrier()
        pltpu.sync_copy(acc_sh.at[pl.ds(sid * RPT, RPT)],
                        out_hbm.at[cid, pl.ds(sid * RPT, RPT)])

    return body(table, gidx4, dst4, zeros_rpt)


def _tc_gather_index(et2, src2):
    BR = 256
    rows = E_PAD // K

    def body(et_ref, src_ref, o_ref):
        o_ref[...] = et_ref[...] * N + src_ref[...]

    return pl.pallas_call(
        body,
        grid=(rows // BR,),
        in_specs=[
            pl.BlockSpec((BR, K), lambda i: (i, 0)),
            pl.BlockSpec((BR, K), lambda i: (i, 0)),
        ],
        out_specs=pl.BlockSpec((BR, K), lambda i: (i, 0)),
        out_shape=jax.ShapeDtypeStruct((rows, K), jnp.int32),
    )(et2, src2)


def _tc_embed(type_ids, emb_pad):
    def body(ids_ref, emb_ref, o_ref):
        ids = ids_ref[...]
        onehot = (ids[:, None] == lax.broadcasted_iota(jnp.int32, (N, 128), 1)
                  ).astype(jnp.float32)
        o_ref[...] = jnp.dot(onehot, emb_ref[...],
                             preferred_element_type=jnp.float32)

    return pl.pallas_call(
        body,
        out_shape=jax.ShapeDtypeStruct((N, H), jnp.float32),
    )(type_ids, emb_pad)


def _tc_transform0(h, W_e, b_e3):
    BN = 2000

    def body(h_ref, w_ref, b_ref, tab_ref):
        hh = h_ref[...]
        for t in range(T):
            tab_ref[t] = (jnp.dot(hh, w_ref[t],
                                  preferred_element_type=jnp.float32)
                          + b_ref[t])

    return pl.pallas_call(
        body,
        grid=(N // BN,),
        in_specs=[
            pl.BlockSpec((BN, H), lambda i: (i, 0)),
            pl.BlockSpec((T, H, H), lambda i: (0, 0, 0)),
            pl.BlockSpec((T, 1, H), lambda i: (0, 0, 0)),
        ],
        out_specs=pl.BlockSpec((T, BN, H), lambda i: (0, i, 0)),
        out_shape=jax.ShapeDtypeStruct((T, N, H), jnp.float32),
    )(h, W_e, b_e3)


def _gru_block(a, hh, wi_ref, wh_ref, bi_ref, bh_ref):
    gi = jnp.dot(a, wi_ref[...], preferred_element_type=jnp.float32) + bi_ref[...]
    gh = jnp.dot(hh, wh_ref[...], preferred_element_type=jnp.float32) + bh_ref[...]
    r = jax.nn.sigmoid(gi[:, :H] + gh[:, :H])
    z = jax.nn.sigmoid(gi[:, H:2 * H] + gh[:, H:2 * H])
    n = jnp.tanh(gi[:, 2 * H:] + r * gh[:, 2 * H:])
    return (1.0 - z) * n + z * hh


def _tc_gru_transform(parts, h, W_ihT, W_hhT, b_ih, b_hh, W_e, b_e3):
    BN = 2000

    def body(p0_ref, p1_ref, h_ref, wi_ref, wh_ref, bi_ref, bh_ref,
             w_ref, b_ref, hn_ref, tab_ref):
        hn = _gru_block(p0_ref[0] + p1_ref[0], h_ref[...],
                        wi_ref, wh_ref, bi_ref, bh_ref)
        hn_ref[...] = hn
        for t in range(T):
            tab_ref[t] = (jnp.dot(hn, w_ref[t],
                                  preferred_element_type=jnp.float32)
                          + b_ref[t])

    return pl.pallas_call(
        body,
        grid=(N // BN,),
        in_specs=[
            pl.BlockSpec((1, BN, H), lambda i: (0, i, 0)),
            pl.BlockSpec((1, BN, H), lambda i: (1, i, 0)),
            pl.BlockSpec((BN, H), lambda i: (i, 0)),
            pl.BlockSpec((H, 3 * H), lambda i: (0, 0)),
            pl.BlockSpec((H, 3 * H), lambda i: (0, 0)),
            pl.BlockSpec((1, 3 * H), lambda i: (0, 0)),
            pl.BlockSpec((1, 3 * H), lambda i: (0, 0)),
            pl.BlockSpec((T, H, H), lambda i: (0, 0, 0)),
            pl.BlockSpec((T, 1, H), lambda i: (0, 0, 0)),
        ],
        out_specs=[
            pl.BlockSpec((BN, H), lambda i: (i, 0)),
            pl.BlockSpec((T, BN, H), lambda i: (0, i, 0)),
        ],
        out_shape=[
            jax.ShapeDtypeStruct((N, H), jnp.float32),
            jax.ShapeDtypeStruct((T, N, H), jnp.float32),
        ],
    )(parts, parts, h, W_ihT, W_hhT, b_ih, b_hh, W_e, b_e3)


def _tc_gru(parts, h, W_ihT, W_hhT, b_ih, b_hh):
    BN = 2000

    def body(p0_ref, p1_ref, h_ref, wi_ref, wh_ref, bi_ref, bh_ref, o_ref):
        o_ref[...] = _gru_block(p0_ref[0] + p1_ref[0], h_ref[...],
                                wi_ref, wh_ref, bi_ref, bh_ref)

    return pl.pallas_call(
        body,
        grid=(N // BN,),
        in_specs=[
            pl.BlockSpec((1, BN, H), lambda i: (0, i, 0)),
            pl.BlockSpec((1, BN, H), lambda i: (1, i, 0)),
            pl.BlockSpec((BN, H), lambda i: (i, 0)),
            pl.BlockSpec((H, 3 * H), lambda i: (0, 0)),
            pl.BlockSpec((H, 3 * H), lambda i: (0, 0)),
            pl.BlockSpec((1, 3 * H), lambda i: (0, 0)),
            pl.BlockSpec((1, 3 * H), lambda i: (0, 0)),
        ],
        out_specs=pl.BlockSpec((BN, H), lambda i: (i, 0)),
        out_shape=jax.ShapeDtypeStruct((N, H), jnp.float32),
    )(parts, parts, h, W_ihT, W_hhT, b_ih, b_hh)


def _tc_pool(h, ann, wg1, wg2, b_gate, wo1, wo2, b_out):
    OUT = b_out.shape[-1]

    def body(h_ref, a_ref, wg1_ref, wg2_ref, bg_ref, wo1_ref, wo2_ref, bo_ref,
             o_ref):
        hh = h_ref[...]
        aa = a_ref[...]
        lg = (jnp.dot(hh, wg1_ref[...], preferred_element_type=jnp.float32)
              + jnp.dot(aa, wg2_ref[...], preferred_element_type=jnp.float32)
              + bg_ref[0, 0])
        m = jnp.max(lg)
        e = jnp.exp(lg - m)
        g = e / jnp.sum(e)
        rh = jnp.sum(g * hh, axis=0, keepdims=True)
        ra = jnp.sum(g * aa, axis=0, keepdims=True)
        o_ref[...] = (jnp.dot(rh, wo1_ref[...], preferred_element_type=jnp.float32)
                      + jnp.dot(ra, wo2_ref[...], preferred_element_type=jnp.float32)
                      + bo_ref[...])

    return pl.pallas_call(
        body,
        out_shape=jax.ShapeDtypeStruct((1, OUT), jnp.float32),
    )(h, ann, wg1, wg2, b_gate, wo1, wo2, b_out)


def kernel(edge_index, etypes, type_ids, emb_table, W_e, b_e, W_ih, W_hh,
           b_ih, b_hh, W_gate, b_gate, W_out, b_out):
    src = edge_index[0]
    dst = edge_index[1]
    pad = E_PAD - E
    pad_r = jnp.arange(pad, dtype=jnp.int32)
    et2 = jnp.concatenate([etypes, pad_r % T]).reshape(E_PAD // K, K)
    src2 = jnp.concatenate([src, (pad_r * 7919) % N]).reshape(E_PAD // K, K)
    gidx = _tc_gather_index(et2, src2).reshape(E_PAD)
    sink = N + (pad_r % (ACC_ROWS - N))
    dst_p = jnp.concatenate([dst, sink])
    gidx_s, dst_s = lax.sort((gidx, dst_p), num_keys=1)
    gidx4 = (gidx_s.reshape(Q, NW, K).transpose(1, 0, 2)
             .reshape(NW, NBLK, BLK, K))
    dst4 = (dst_s.reshape(Q, NW, K).transpose(1, 0, 2)
            .reshape(NW, NBLK, BLK, K))
    zeros_rpt = jnp.zeros((RPT, H), jnp.float32)

    W_ihT = W_ih.T
    W_hhT = W_hh.T
    b_ih2 = b_ih.reshape(1, 3 * H)
    b_hh2 = b_hh.reshape(1, 3 * H)
    b_e3 = b_e.reshape(T, 1, H)
    wg1 = W_gate[:H]
    wg2 = W_gate[H:]
    wo1 = W_out[:H]
    wo2 = W_out[H:]
    bg2 = b_gate.reshape(1, 1)
    bo2 = b_out.reshape(1, -1)

    emb_pad = jnp.zeros((128, H), jnp.float32).at[:100].set(emb_table)
    ann = _tc_embed(type_ids, emb_pad)
    table = _tc_transform0(ann, W_e, b_e3)
    h = ann
    for step in range(N_STEPS):
        parts = _edge_aggregate(table.reshape(T * N, H), gidx4, dst4,
                                zeros_rpt)
        if step < N_STEPS - 1:
            h, table = _tc_gru_transform(parts, h, W_ihT, W_hhT, b_ih2,
                                         b_hh2, W_e, b_e3)
        else:
            h = _tc_gru(parts, h, W_ihT, W_hhT, b_ih2, b_hh2)
    return _tc_pool(h, ann, wg1, wg2, bg2, wo1, wo2, bo2)


# speedup vs baseline: 2.5962x; 1.0455x over previous
import functools

import jax
import jax.numpy as jnp
from jax import lax
from jax.experimental import pallas as pl
from jax.experimental.pallas import tpu as pltpu
from jax.experimental.pallas import tpu_sc as plsc

N = 10000
E = 320000
H = 128
T = 3
N_STEPS = 6

NC = 2
NS = 16
NW = NC * NS

K = 64
Q = 160
BLK = 32
NBLK = Q // BLK
EPW = Q * K
E_PAD = EPW * NW
DEPTH = 4

RPT = 632
ACC_ROWS = RPT * NS


def _edge_aggregate(table, gidx4, dst4, zeros_rpt):
    mesh = plsc.VectorSubcoreMesh(core_axis_name="c", subcore_axis_name="s")

    @functools.partial(
        pl.kernel,
        out_type=jax.ShapeDtypeStruct((NC, ACC_ROWS, H), jnp.float32),
        mesh=mesh,
        scratch_types=[
            pltpu.VMEM((BLK, K), jnp.int32),
            pltpu.VMEM((BLK, K), jnp.int32),
            pltpu.VMEM((BLK, K), jnp.int32),
            pltpu.VMEM((BLK, K), jnp.int32),
            pltpu.VMEM((K, H), jnp.float32),
            pltpu.VMEM((K, H), jnp.float32),
            pltpu.VMEM((K, H), jnp.float32),
            pltpu.VMEM((K, H), jnp.float32),
            pltpu.VMEM_SHARED((ACC_ROWS, H), jnp.float32),
            pltpu.SemaphoreType.DMA,
            pltpu.SemaphoreType.DMA,
            pltpu.SemaphoreType.DMA,
            pltpu.SemaphoreType.DMA,
            pltpu.SemaphoreType.DMA,
        ],
    )
    def body(table_hbm, gidx_hbm, dst_hbm, z_hbm, out_hbm,
             g0, g1, d0, d1, rows0, rows1, rows2, rows3, acc_sh,
             sem0, sem1, sem2, sem3, semi):
        cid = lax.axis_index("c")
        sid = lax.axis_index("s")
        wid = cid * NS + sid

        pltpu.sync_copy(gidx_hbm.at[wid, 0], g0)
        pltpu.sync_copy(dst_hbm.at[wid, 0], d0)
        pltpu.sync_copy(z_hbm, acc_sh.at[pl.ds(sid * RPT, RPT)])
        plsc.subcore_barrier()

        gbufs = (g0, g1)
        dbufs = (d0, d1)
        rbufs = (rows0, rows1, rows2, rows3)
        sems = (sem0, sem1, sem2, sem3)
        for jb in range(NBLK):
            ga, da = gbufs[jb % 2], dbufs[jb % 2]
            gn, dn = gbufs[1 - jb % 2], dbufs[1 - jb % 2]
            if jb < NBLK - 1:
                pltpu.async_copy(gidx_hbm.at[wid, jb + 1], gn, semi)
                pltpu.async_copy(dst_hbm.at[wid, jb + 1], dn, semi)

            for k in range(DEPTH):
                pltpu.async_copy(table_hbm.at[ga.at[k]], rbufs[k], sems[k])

            def group(g, _, ga=ga, da=da):
                c0 = DEPTH * g
                for k in range(DEPTH):
                    pltpu.make_async_copy(table_hbm.at[ga.at[c0 + k]],
                                          rbufs[k], sems[k]).wait()
                    pltpu.sync_copy(rbufs[k], acc_sh.at[da.at[c0 + k]],
                                    add=True)

                    @pl.when(g < BLK // DEPTH - 1)
                    def _(k=k):
                        pltpu.async_copy(table_hbm.at[ga.at[c0 + DEPTH + k]],
                                         rbufs[k], sems[k])
                return 0

            lax.fori_loop(0, BLK // DEPTH, group, 0)
            if jb < NBLK - 1:
                pltpu.make_async_copy(gidx_hbm.at[wid, jb + 1], gn, semi).wait()
                pltpu.make_async_copy(dst_hbm.at[wid, jb + 1], dn, semi).wait()

        plsc.subcore_barrier()
        pltpu.sync_copy(acc_sh.at[pl.ds(sid * RPT, RPT)],
                        out_hbm.at[cid, pl.ds(sid * RPT, RPT)])

    return body(table, gidx4, dst4, zeros_rpt)


def _tc_gather_index(et2, src2):
    BR = 256
    rows = E_PAD // K

    def body(et_ref, src_ref, o_ref):
        o_ref[...] = et_ref[...] * N + src_ref[...]

    return pl.pallas_call(
        body,
        grid=(rows // BR,),
        in_specs=[
            pl.BlockSpec((BR, K), lambda i: (i, 0)),
            pl.BlockSpec((BR, K), lambda i: (i, 0)),
        ],
        out_specs=pl.BlockSpec((BR, K), lambda i: (i, 0)),
        out_shape=jax.ShapeDtypeStruct((rows, K), jnp.int32),
    )(et2, src2)


def _tc_embed(type_ids, emb_pad):
    def body(ids_ref, emb_ref, o_ref):
        ids = ids_ref[...]
        onehot = (ids[:, None] == lax.broadcasted_iota(jnp.int32, (N, 128), 1)
                  ).astype(jnp.float32)
        o_ref[...] = jnp.dot(onehot, emb_ref[...],
                             preferred_element_type=jnp.float32)

    return pl.pallas_call(
        body,
        out_shape=jax.ShapeDtypeStruct((N, H), jnp.float32),
    )(type_ids, emb_pad)


def _tc_transform0(h, W_e, b_e3):
    BN = 2000

    def body(h_ref, w_ref, b_ref, tab_ref):
        hh = h_ref[...]
        for t in range(T):
            tab_ref[t] = (jnp.dot(hh, w_ref[t],
                                  preferred_element_type=jnp.float32)
                          + b_ref[t])

    return pl.pallas_call(
        body,
        grid=(N // BN,),
        in_specs=[
            pl.BlockSpec((BN, H), lambda i: (i, 0)),
            pl.BlockSpec((T, H, H), lambda i: (0, 0, 0)),
            pl.BlockSpec((T, 1, H), lambda i: (0, 0, 0)),
        ],
        out_specs=pl.BlockSpec((T, BN, H), lambda i: (0, i, 0)),
        out_shape=jax.ShapeDtypeStruct((T, N, H), jnp.float32),
    )(h, W_e, b_e3)


def _gru_block(a, hh, wi_ref, wh_ref, bi_ref, bh_ref):
    gi = jnp.dot(a, wi_ref[...], preferred_element_type=jnp.float32) + bi_ref[...]
    gh = jnp.dot(hh, wh_ref[...], preferred_element_type=jnp.float32) + bh_ref[...]
    r = jax.nn.sigmoid(gi[:, :H] + gh[:, :H])
    z = jax.nn.sigmoid(gi[:, H:2 * H] + gh[:, H:2 * H])
    n = jnp.tanh(gi[:, 2 * H:] + r * gh[:, 2 * H:])
    return (1.0 - z) * n + z * hh


def _tc_gru_transform(parts, h, W_ihT, W_hhT, b_ih, b_hh, W_e, b_e3):
    BN = 2000

    def body(p0_ref, p1_ref, h_ref, wi_ref, wh_ref, bi_ref, bh_ref,
             w_ref, b_ref, hn_ref, tab_ref):
        hn = _gru_block(p0_ref[0] + p1_ref[0], h_ref[...],
                        wi_ref, wh_ref, bi_ref, bh_ref)
        hn_ref[...] = hn
        for t in range(T):
            tab_ref[t] = (jnp.dot(hn, w_ref[t],
                                  preferred_element_type=jnp.float32)
                          + b_ref[t])

    return pl.pallas_call(
        body,
        grid=(N // BN,),
        in_specs=[
            pl.BlockSpec((1, BN, H), lambda i: (0, i, 0)),
            pl.BlockSpec((1, BN, H), lambda i: (1, i, 0)),
            pl.BlockSpec((BN, H), lambda i: (i, 0)),
            pl.BlockSpec((H, 3 * H), lambda i: (0, 0)),
            pl.BlockSpec((H, 3 * H), lambda i: (0, 0)),
            pl.BlockSpec((1, 3 * H), lambda i: (0, 0)),
            pl.BlockSpec((1, 3 * H), lambda i: (0, 0)),
            pl.BlockSpec((T, H, H), lambda i: (0, 0, 0)),
            pl.BlockSpec((T, 1, H), lambda i: (0, 0, 0)),
        ],
        out_specs=[
            pl.BlockSpec((BN, H), lambda i: (i, 0)),
            pl.BlockSpec((T, BN, H), lambda i: (0, i, 0)),
        ],
        out_shape=[
            jax.ShapeDtypeStruct((N, H), jnp.float32),
            jax.ShapeDtypeStruct((T, N, H), jnp.float32),
        ],
    )(parts, parts, h, W_ihT, W_hhT, b_ih, b_hh, W_e, b_e3)


def _tc_gru(parts, h, W_ihT, W_hhT, b_ih, b_hh):
    BN = 2000

    def body(p0_ref, p1_ref, h_ref, wi_ref, wh_ref, bi_ref, bh_ref, o_ref):
        o_ref[...] = _gru_block(p0_ref[0] + p1_ref[0], h_ref[...],
                                wi_ref, wh_ref, bi_ref, bh_ref)

    return pl.pallas_call(
        body,
        grid=(N // BN,),
        in_specs=[
            pl.BlockSpec((1, BN, H), lambda i: (0, i, 0)),
            pl.BlockSpec((1, BN, H), lambda i: (1, i, 0)),
            pl.BlockSpec((BN, H), lambda i: (i, 0)),
            pl.BlockSpec((H, 3 * H), lambda i: (0, 0)),
            pl.BlockSpec((H, 3 * H), lambda i: (0, 0)),
            pl.BlockSpec((1, 3 * H), lambda i: (0, 0)),
            pl.BlockSpec((1, 3 * H), lambda i: (0, 0)),
        ],
        out_specs=pl.BlockSpec((BN, H), lambda i: (i, 0)),
        out_shape=jax.ShapeDtypeStruct((N, H), jnp.float32),
    )(parts, parts, h, W_ihT, W_hhT, b_ih, b_hh)


def _tc_pool(h, ann, wg1, wg2, b_gate, wo1, wo2, b_out):
    OUT = b_out.shape[-1]

    def body(h_ref, a_ref, wg1_ref, wg2_ref, bg_ref, wo1_ref, wo2_ref, bo_ref,
             o_ref):
        hh = h_ref[...]
        aa = a_ref[...]
        lg = (jnp.dot(hh, wg1_ref[...], preferred_element_type=jnp.float32)
              + jnp.dot(aa, wg2_ref[...], preferred_element_type=jnp.float32)
              + bg_ref[0, 0])
        m = jnp.max(lg)
        e = jnp.exp(lg - m)
        g = e / jnp.sum(e)
        rh = jnp.sum(g * hh, axis=0, keepdims=True)
        ra = jnp.sum(g * aa, axis=0, keepdims=True)
        o_ref[...] = (jnp.dot(rh, wo1_ref[...], preferred_element_type=jnp.float32)
                      + jnp.dot(ra, wo2_ref[...], preferred_element_type=jnp.float32)
                      + bo_ref[...])

    return pl.pallas_call(
        body,
        out_shape=jax.ShapeDtypeStruct((1, OUT), jnp.float32),
    )(h, ann, wg1, wg2, b_gate, wo1, wo2, b_out)


def kernel(edge_index, etypes, type_ids, emb_table, W_e, b_e, W_ih, W_hh,
           b_ih, b_hh, W_gate, b_gate, W_out, b_out):
    src = edge_index[0]
    dst = edge_index[1]
    pad = E_PAD - E
    pad_r = jnp.arange(pad, dtype=jnp.int32)
    et2 = jnp.concatenate([etypes, pad_r % T]).reshape(E_PAD // K, K)
    src2 = jnp.concatenate([src, (pad_r * 7919) % N]).reshape(E_PAD // K, K)
    gidx = _tc_gather_index(et2, src2).reshape(E_PAD)
    sink = N + (pad_r % (ACC_ROWS - N))
    dst_p = jnp.concatenate([dst, sink])
    gidx_s, dst_s = lax.sort((gidx, dst_p), num_keys=1)
    gidx4 = (gidx_s.reshape(Q, NW, K).transpose(1, 0, 2)
             .reshape(NW, NBLK, BLK, K))
    dst4 = (dst_s.reshape(Q, NW, K).transpose(1, 0, 2)
            .reshape(NW, NBLK, BLK, K))
    zeros_rpt = jnp.zeros((RPT, H), jnp.float32)

    W_ihT = W_ih.T
    W_hhT = W_hh.T
    b_ih2 = b_ih.reshape(1, 3 * H)
    b_hh2 = b_hh.reshape(1, 3 * H)
    b_e3 = b_e.reshape(T, 1, H)
    wg1 = W_gate[:H]
    wg2 = W_gate[H:]
    wo1 = W_out[:H]
    wo2 = W_out[H:]
    bg2 = b_gate.reshape(1, 1)
    bo2 = b_out.reshape(1, -1)

    emb_pad = jnp.zeros((128, H), jnp.float32).at[:100].set(emb_table)
    ann = _tc_embed(type_ids, emb_pad)
    table = _tc_transform0(ann, W_e, b_e3)
    h = ann
    for step in range(N_STEPS):
        parts = _edge_aggregate(table.reshape(T * N, H), gidx4, dst4,
                                zeros_rpt)
        if step < N_STEPS - 1:
            h, table = _tc_gru_transform(parts, h, W_ihT, W_hhT, b_ih2,
                                         b_hh2, W_e, b_e3)
        else:
            h = _tc_gru(parts, h, W_ihT, W_hhT, b_ih2, b_hh2)
    return _tc_pool(h, ann, wg1, wg2, bg2, wo1, wo2, bo2)
